# baseline (speedup 1.0000x reference)
import functools

import jax
import jax.numpy as jnp
from jax import lax
from jax.experimental import pallas as pl
from jax.experimental.pallas import tpu as pltpu
from jax.experimental.pallas import tpu_sc as plsc

VOCAB = 100000
D = 768
B = 4
S = 2048
N = B * S
LANES = 16
VPD = D // LANES

_info = plsc.get_sparse_core_info()
NC, NS = _info.num_cores, _info.num_subcores
NW = NC * NS
S_PER_W = S // NW
SP_CHUNK = 8
CHUNK = B * SP_CHUNK
NBUF = 3
NCHUNK = S_PER_W // SP_CHUNK


def _body(ids_hbm, table_hbm, pos_hbm, dim_hbm, out_hbm,
          idx_v, pos_v, dim_v,
          rows0, rows1, rows2,
          gsem0, gsem1, gsem2, osem0, osem1, osem2, isem):
    rows = (rows0, rows1, rows2)
    gsem = (gsem0, gsem1, gsem2)
    osem = (osem0, osem1, osem2)

    wid = lax.axis_index("s") * NC + lax.axis_index("c")
    s0 = wid * S_PER_W

    idd = []
    for c in range(NCHUNK):
        for b in range(B):
            idd.append(pltpu.async_copy(
                ids_hbm.at[pl.ds(b * S + s0 + c * SP_CHUNK, SP_CHUNK)],
                idx_v.at[pl.ds(c * CHUNK + b * SP_CHUNK, SP_CHUNK)], isem))
    for d in idd:
        d.wait()

    HALF = CHUNK // 2

    def start_gather(c):
        bid = c % NBUF
        return [
            pltpu.async_copy(
                table_hbm.at[idx_v.at[pl.ds(c * CHUNK + h * HALF, HALF)]],
                rows[bid].at[pl.ds(h * HALF, HALF)], gsem[bid])
            for h in range(2)
        ]

    gd = [None] * NBUF
    od = [[] for _ in range(NBUF)]
    gd[0] = start_gather(0)
    gd[1] = start_gather(1)

    pltpu.sync_copy(dim_hbm, dim_v)
    pltpu.sync_copy(pos_hbm.at[pl.ds(s0, S_PER_W)], pos_v)

    for j in range(VPD):
        sl = pl.ds(j * LANES, LANES)
        acc = dim_v[0, sl] + dim_v[1, sl] + dim_v[2, sl] + dim_v[3, sl] + dim_v[4, sl]
        m = acc * 0.2

        def fold_body(ro, carry, sl=sl, m=m):
            for rr in range(8):
                r = ro * 8 + rr
                plsc.addupdate(pos_v.at[r, sl], m)
            return carry

        lax.fori_loop(0, S_PER_W // 8, fold_body, 0)

    for c in range(NCHUNK):
        bid = c % NBUF
        nxt = c + 2
        if nxt < NCHUNK:
            nb = nxt % NBUF
            for d in od[nb]:
                d.wait()
            od[nb] = []
            gd[nb] = start_gather(nxt)
        for d in gd[bid]:
            d.wait()

        pbase = c * SP_CHUNK

        def add_row(r, carry):
            def add_jg(jg, carry2):
                for jj in range(4):
                    sl = pl.ds((jg * 4 + jj) * LANES, LANES)
                    p = pos_v[pbase + r, sl]
                    for b in range(B):
                        row = b * SP_CHUNK + r
                        plsc.addupdate(rows[bid].at[row, sl], p)
                return carry2

            lax.fori_loop(0, VPD // 4, add_jg, 0)
            return carry

        lax.fori_loop(0, SP_CHUNK, add_row, 0)

        for d in od[bid]:
            d.wait()
        od[bid] = [
            pltpu.async_copy(
                rows[bid].at[pl.ds(b * SP_CHUNK, SP_CHUNK)],
                out_hbm.at[pl.ds(b * S + s0 + c * SP_CHUNK, SP_CHUNK)],
                osem[bid])
            for b in range(B)
        ]

    for bl in od:
        for d in bl:
            d.wait()


@jax.jit
def _run(ids_flat, table, pos_enc, dim_emb):
    mesh = plsc.VectorSubcoreMesh(core_axis_name="c", subcore_axis_name="s")
    kern = functools.partial(
        pl.kernel,
        out_type=jax.ShapeDtypeStruct((N, D), jnp.float32),
        mesh=mesh,
        scratch_types=[
            pltpu.VMEM((B * S_PER_W,), jnp.int32),
            pltpu.VMEM((S_PER_W, D), jnp.float32),
            pltpu.VMEM((5, D), jnp.float32),
            pltpu.VMEM((CHUNK, D), jnp.float32),
            pltpu.VMEM((CHUNK, D), jnp.float32),
            pltpu.VMEM((CHUNK, D), jnp.float32),
            pltpu.SemaphoreType.DMA,
            pltpu.SemaphoreType.DMA,
            pltpu.SemaphoreType.DMA,
            pltpu.SemaphoreType.DMA,
            pltpu.SemaphoreType.DMA,
            pltpu.SemaphoreType.DMA,
            pltpu.SemaphoreType.DMA,
        ],
    )(_body)
    return kern(ids_flat, table, pos_enc, dim_emb)


def kernel(input_ids, table, pos_enc, dim_emb):
    ids_flat = input_ids.reshape(N).astype(jnp.int32)
    out = _run(ids_flat, table, pos_enc, dim_emb)
    return out.reshape(B, S, D)


# speedup vs baseline: 1.1775x; 1.1775x over previous
import functools

import jax
import jax.numpy as jnp
from jax import lax
from jax.experimental import pallas as pl
from jax.experimental.pallas import tpu as pltpu
from jax.experimental.pallas import tpu_sc as plsc

VOCAB = 100000
D = 768
B = 4
S = 2048
N = B * S
LANES = 16
VPD = D // LANES

_info = plsc.get_sparse_core_info()
NC, NS = _info.num_cores, _info.num_subcores
NW = NC * NS
S_PER_W = S // NW
SP_CHUNK = 8
CHUNK = B * SP_CHUNK
NBUF = 3
NCHUNK = S_PER_W // SP_CHUNK


def _body(ids_hbm, table_hbm, pos_hbm, dim_hbm, out_hbm,
          idx_v, pos_v, dim_v,
          rows0, rows1, rows2,
          gsem0, gsem1, gsem2, osem0, osem1, osem2, isem):
    rows = (rows0, rows1, rows2)
    gsem = (gsem0, gsem1, gsem2)
    osem = (osem0, osem1, osem2)

    wid = lax.axis_index("s") * NC + lax.axis_index("c")
    s0 = wid * S_PER_W

    idd = []
    for c in range(NCHUNK):
        for b in range(B):
            idd.append(pltpu.async_copy(
                ids_hbm.at[pl.ds(b * S + s0 + c * SP_CHUNK, SP_CHUNK)],
                idx_v.at[pl.ds(c * CHUNK + b * SP_CHUNK, SP_CHUNK)], isem))
    for d in idd:
        d.wait()

    HALF = CHUNK // 2

    def start_gather(c):
        bid = c % NBUF
        return [
            pltpu.async_copy(
                table_hbm.at[idx_v.at[pl.ds(c * CHUNK + h * HALF, HALF)]],
                rows[bid].at[pl.ds(h * HALF, HALF)], gsem[bid])
            for h in range(2)
        ]

    gd = [None] * NBUF
    od = [[] for _ in range(NBUF)]
    gd[0] = start_gather(0)
    gd[1] = start_gather(1)

    pltpu.sync_copy(dim_hbm, dim_v)
    pltpu.sync_copy(pos_hbm.at[pl.ds(s0, S_PER_W)], pos_v)

    def fold_col(j, carry):
        sl = pl.ds(j * LANES, LANES)
        acc = dim_v[0, sl] + dim_v[1, sl] + dim_v[2, sl] + dim_v[3, sl] + dim_v[4, sl]
        m = acc * 0.2

        def fold_body(ro, carry2):
            for rr in range(8):
                r = ro * 8 + rr
                plsc.addupdate(pos_v.at[r, sl], m)
            return carry2

        lax.fori_loop(0, S_PER_W // 8, fold_body, 0)
        return carry

    lax.fori_loop(0, VPD, fold_col, 0)

    for c in range(NCHUNK):
        bid = c % NBUF
        nxt = c + 2
        if nxt < NCHUNK:
            nb = nxt % NBUF
            for d in od[nb]:
                d.wait()
            od[nb] = []
            gd[nb] = start_gather(nxt)
        for d in gd[bid]:
            d.wait()

        pbase = c * SP_CHUNK

        def add_row(r, carry):
            for j in range(VPD):
                sl = pl.ds(j * LANES, LANES)
                p = pos_v[pbase + r, sl]
                for b in range(B):
                    row = b * SP_CHUNK + r
                    plsc.addupdate(rows[bid].at[row, sl], p)
            return carry

        lax.fori_loop(0, SP_CHUNK, add_row, 0)

        for d in od[bid]:
            d.wait()
        od[bid] = [
            pltpu.async_copy(
                rows[bid].at[pl.ds(b * SP_CHUNK, SP_CHUNK)],
                out_hbm.at[pl.ds(b * S + s0 + c * SP_CHUNK, SP_CHUNK)],
                osem[bid])
            for b in range(B)
        ]

    for bl in od:
        for d in bl:
            d.wait()


@jax.jit
def _run(ids_flat, table, pos_enc, dim_emb):
    mesh = plsc.VectorSubcoreMesh(core_axis_name="c", subcore_axis_name="s")
    kern = functools.partial(
        pl.kernel,
        out_type=jax.ShapeDtypeStruct((N, D), jnp.float32),
        mesh=mesh,
        scratch_types=[
            pltpu.VMEM((B * S_PER_W,), jnp.int32),
            pltpu.VMEM((S_PER_W, D), jnp.float32),
            pltpu.VMEM((5, D), jnp.float32),
            pltpu.VMEM((CHUNK, D), jnp.float32),
            pltpu.VMEM((CHUNK, D), jnp.float32),
            pltpu.VMEM((CHUNK, D), jnp.float32),
            pltpu.SemaphoreType.DMA,
            pltpu.SemaphoreType.DMA,
            pltpu.SemaphoreType.DMA,
            pltpu.SemaphoreType.DMA,
            pltpu.SemaphoreType.DMA,
            pltpu.SemaphoreType.DMA,
            pltpu.SemaphoreType.DMA,
        ],
    )(_body)
    return kern(ids_flat, table, pos_enc, dim_emb)


def kernel(input_ids, table, pos_enc, dim_emb):
    ids_flat = input_ids.reshape(N).astype(jnp.int32)
    out = _run(ids_flat, table, pos_enc, dim_emb)
    return out.reshape(B, S, D)
